# baseline (speedup 1.0000x reference)
import functools

import jax
import jax.numpy as jnp
import numpy as np
from jax import lax
from jax.experimental import pallas as pl
from jax.experimental.pallas import tpu as pltpu
from jax.experimental.pallas import tpu_sc as plsc

_ROWS = 128
_COLS = 32768
_WORDS = _COLS // 2
_NUM_CORES = 2
_NUM_SUBCORES = 16
_NUM_WORKERS = _NUM_CORES * _NUM_SUBCORES
_ROWS_PER_WORKER = _ROWS // _NUM_WORKERS
_LANES = 16
_ITERS = _WORDS // _LANES

_SIGN2 = np.int32(-2147450880)
_HI16 = np.int32(-65536)
_ONE2 = np.int32(0x00010001)
_INT32_MIN = np.int32(-(2**31))


def _row_argmax(rows_v, r):
    def body(i, carry):
        acc_l, acc_h, inv_l, inv_h = carry
        v = rows_v[r, pl.ds(i * _LANES, _LANES)]
        m = (v & _SIGN2) - ((v >> 15) & _ONE2)
        kk = v ^ m
        c_l = (kk << 16) | inv_l
        c_h = (kk & _HI16) | inv_h
        acc_l = jnp.maximum(acc_l, c_l)
        acc_h = jnp.maximum(acc_h, c_h)
        return acc_l, acc_h, inv_l - 32, inv_h - 32

    inv_l0 = np.int32(32767) - 2 * lax.iota(jnp.int32, _LANES)
    acc0 = jnp.full((_LANES,), _INT32_MIN, jnp.int32)
    acc_l, acc_h, _, _ = lax.fori_loop(
        0, _ITERS, body, (acc0, acc0, inv_l0, inv_l0 - 1)
    )
    acc = jnp.maximum(acc_l, acc_h)
    best = acc[0]
    for j in range(1, _LANES):
        best = jnp.maximum(best, acc[j])
    return np.int32(32767) - (best & np.int32(0xFFFF))


@functools.partial(
    pl.kernel,
    mesh=plsc.VectorSubcoreMesh(core_axis_name="c", subcore_axis_name="s"),
    out_type=jax.ShapeDtypeStruct((_NUM_WORKERS, 16), jnp.int32),
    scratch_types=[
        pltpu.VMEM((_ROWS_PER_WORKER, _WORDS), jnp.int32),
        pltpu.VMEM((16,), jnp.int32),
    ],
)
def _sc_argmax(x_hbm, out_hbm, rows_v, res_v):
    wid = lax.axis_index("s") * _NUM_CORES + lax.axis_index("c")
    base = wid * _ROWS_PER_WORKER
    pltpu.sync_copy(x_hbm.at[pl.ds(base, _ROWS_PER_WORKER)], rows_v)
    lane16 = lax.iota(jnp.int32, 16)
    res = jnp.zeros((16,), jnp.int32)
    for r in range(_ROWS_PER_WORKER):
        bi = _row_argmax(rows_v, r)
        res = jnp.where(lane16 == r, bi, res)
    res_v[...] = res
    pltpu.sync_copy(res_v, out_hbm.at[wid])


def kernel(input_tensor, dim):
    del dim
    packed = lax.bitcast_convert_type(
        input_tensor.reshape(_ROWS, _WORDS, 2), jnp.int32
    )
    out = _sc_argmax(packed)
    return out[:, :_ROWS_PER_WORKER].reshape(_ROWS).astype(jnp.int64)


# speedup vs baseline: 1.0080x; 1.0080x over previous
import functools

import jax
import jax.numpy as jnp
import numpy as np
from jax import lax
from jax.experimental import pallas as pl
from jax.experimental.pallas import tpu as pltpu
from jax.experimental.pallas import tpu_sc as plsc

_ROWS = 128
_COLS = 32768
_WORDS = _COLS // 2
_NUM_CORES = 2
_NUM_SUBCORES = 16
_NUM_WORKERS = _NUM_CORES * _NUM_SUBCORES
_ROWS_PER_WORKER = _ROWS // _NUM_WORKERS
_LANES = 16
_ITERS = _WORDS // _LANES

_SIGN2 = np.int32(-2147450880)
_HI16 = np.int32(-65536)
_ONE2 = np.int32(0x00010001)
_INT32_MIN = np.int32(-(2**31))


def _row_argmax(rows_v, r):
    def body(i, carry):
        acc_l, acc_h, inv_l, inv_h = carry
        v = rows_v[r, pl.ds(i * _LANES, _LANES)]
        m = (v & _SIGN2) - ((v >> 15) & _ONE2)
        kk = v ^ m
        c_l = (kk << 16) | inv_l
        c_h = (kk & _HI16) | inv_h
        acc_l = jnp.maximum(acc_l, c_l)
        acc_h = jnp.maximum(acc_h, c_h)
        return acc_l, acc_h, inv_l - 32, inv_h - 32

    inv_l0 = np.int32(32767) - 2 * lax.iota(jnp.int32, _LANES)
    acc0 = jnp.full((_LANES,), _INT32_MIN, jnp.int32)
    acc_l, acc_h, _, _ = lax.fori_loop(
        0, _ITERS, body, (acc0, acc0, inv_l0, inv_l0 - 1), unroll=8
    )
    acc = jnp.maximum(acc_l, acc_h)
    best = acc[0]
    for j in range(1, _LANES):
        best = jnp.maximum(best, acc[j])
    return np.int32(32767) - (best & np.int32(0xFFFF))


@functools.partial(
    pl.kernel,
    mesh=plsc.VectorSubcoreMesh(core_axis_name="c", subcore_axis_name="s"),
    out_type=jax.ShapeDtypeStruct((_NUM_WORKERS, 16), jnp.int32),
    scratch_types=[
        pltpu.VMEM((_ROWS_PER_WORKER, _WORDS), jnp.int32),
        pltpu.VMEM((16,), jnp.int32),
    ],
)
def _sc_argmax(x_hbm, out_hbm, rows_v, res_v):
    wid = lax.axis_index("s") * _NUM_CORES + lax.axis_index("c")
    base = wid * _ROWS_PER_WORKER
    pltpu.sync_copy(x_hbm.at[pl.ds(base, _ROWS_PER_WORKER)], rows_v)
    lane16 = lax.iota(jnp.int32, 16)
    res = jnp.zeros((16,), jnp.int32)
    for r in range(_ROWS_PER_WORKER):
        bi = _row_argmax(rows_v, r)
        res = jnp.where(lane16 == r, bi, res)
    res_v[...] = res
    pltpu.sync_copy(res_v, out_hbm.at[wid])


def kernel(input_tensor, dim):
    del dim
    packed = lax.bitcast_convert_type(
        input_tensor.reshape(_ROWS, _WORDS, 2), jnp.int32
    )
    out = _sc_argmax(packed)
    return out[:, :_ROWS_PER_WORKER].reshape(_ROWS).astype(jnp.int64)


# speedup vs baseline: 3.9514x; 3.9200x over previous
import functools

import jax
import jax.numpy as jnp
import numpy as np
from jax import lax
from jax.experimental import pallas as pl
from jax.experimental.pallas import tpu as pltpu
from jax.experimental.pallas import tpu_sc as plsc

_ROWS = 128
_COLS = 32768
_NUM_CORES = 2
_NUM_SUBCORES = 16
_NUM_WORKERS = _NUM_CORES * _NUM_SUBCORES
_PACKED_PER_WORKER = 2
_LANES = 16
_ITERS = _COLS // _LANES

_SIGN2 = np.int32(-2147450880)
_HI16 = np.int32(-65536)
_ONE2 = np.int32(0x00010001)
_INT32_MIN = np.int32(-(2**31))


def _packed_row_argmax(rows_v, rr):
    def body(i, carry):
        acc_l, acc_h, inv_l, inv_h = carry
        v = rows_v[rr, pl.ds(i * _LANES, _LANES)]
        m = (v & _SIGN2) - ((v >> 15) & _ONE2)
        kk = v ^ m
        c_l = (kk << 16) | inv_l
        c_h = (kk & _HI16) | inv_h
        acc_l = jnp.maximum(acc_l, c_l)
        acc_h = jnp.maximum(acc_h, c_h)
        return acc_l, acc_h, inv_l - _LANES, inv_h - _LANES

    inv0 = np.int32(32767) - lax.iota(jnp.int32, _LANES)
    acc0 = jnp.full((_LANES,), _INT32_MIN, jnp.int32)
    acc_l, acc_h, _, _ = lax.fori_loop(
        0, _ITERS, body, (acc0, acc0, inv0, inv0), unroll=8
    )

    def merge(acc):
        best = acc[0]
        for j in range(1, _LANES):
            best = jnp.maximum(best, acc[j])
        return np.int32(32767) - (best & np.int32(0xFFFF))

    return merge(acc_l), merge(acc_h)


@functools.partial(
    pl.kernel,
    mesh=plsc.VectorSubcoreMesh(core_axis_name="c", subcore_axis_name="s"),
    out_type=jax.ShapeDtypeStruct((_NUM_WORKERS, 16), jnp.int32),
    scratch_types=[
        pltpu.VMEM((_PACKED_PER_WORKER, _COLS), jnp.int32),
        pltpu.VMEM((16,), jnp.int32),
    ],
)
def _sc_argmax(x_hbm, out_hbm, rows_v, res_v):
    wid = lax.axis_index("s") * _NUM_CORES + lax.axis_index("c")
    x32 = x_hbm.bitcast(jnp.int32)
    pltpu.sync_copy(
        x32.at[pl.ds(wid * _PACKED_PER_WORKER, _PACKED_PER_WORKER)], rows_v
    )
    lane16 = lax.iota(jnp.int32, 16)
    res = jnp.zeros((16,), jnp.int32)
    for rr in range(_PACKED_PER_WORKER):
        bi_even, bi_odd = _packed_row_argmax(rows_v, rr)
        res = jnp.where(lane16 == 2 * rr, bi_even, res)
        res = jnp.where(lane16 == 2 * rr + 1, bi_odd, res)
    res_v[...] = res
    pltpu.sync_copy(res_v, out_hbm.at[wid])


def kernel(input_tensor, dim):
    del dim
    bits = lax.bitcast_convert_type(input_tensor, jnp.int16)
    out = _sc_argmax(bits)
    return out[:, : 2 * _PACKED_PER_WORKER].reshape(_ROWS).astype(jnp.int64)


# speedup vs baseline: 5.0139x; 1.2689x over previous
import functools

import jax
import jax.numpy as jnp
import numpy as np
from jax import lax
from jax.experimental import pallas as pl
from jax.experimental.pallas import tpu as pltpu
from jax.experimental.pallas import tpu_sc as plsc

_ROWS = 128
_COLS = 32768
_NUM_CORES = 2
_NUM_SUBCORES = 16
_NUM_WORKERS = _NUM_CORES * _NUM_SUBCORES
_PACKED_PER_WORKER = 2
_LANES = 16
_HALF = _COLS // 2
_CHUNK_ITERS = _HALF // _LANES

_SIGN2 = np.int32(-2147450880)
_HI16 = np.int32(-65536)
_ONE2 = np.int32(0x00010001)
_INT32_MIN = np.int32(-(2**31))


def _scan_chunk(buf, col0, carry):
    def body(i, carry):
        acc_l, acc_h, inv_l, inv_h = carry
        v = buf[pl.ds(i * _LANES, _LANES)]
        c_l = (v << 16) | inv_l
        c_h = (v & _HI16) | inv_h
        acc_l = jnp.maximum(acc_l, c_l)
        acc_h = jnp.maximum(acc_h, c_h)
        return acc_l, acc_h, inv_l - _LANES, inv_h - _LANES

    inv0 = (np.int32(32767) - col0) - lax.iota(jnp.int32, _LANES)
    acc_l, acc_h, _, _ = lax.fori_loop(
        0, _CHUNK_ITERS, body, (*carry, inv0, inv0), unroll=8
    )
    return acc_l, acc_h


def _scan_row_exact(rows_v, rr):
    def body(i, carry):
        acc_l, acc_h, inv_l, inv_h = carry
        v = rows_v[rr, pl.ds(i * _LANES, _LANES)]
        m = (v & _SIGN2) - ((v >> 15) & _ONE2)
        kk = v ^ m
        c_l = (kk << 16) | inv_l
        c_h = (kk & _HI16) | inv_h
        acc_l = jnp.maximum(acc_l, c_l)
        acc_h = jnp.maximum(acc_h, c_h)
        return acc_l, acc_h, inv_l - _LANES, inv_h - _LANES

    inv0 = np.int32(32767) - lax.iota(jnp.int32, _LANES)
    acc0 = jnp.full((_LANES,), _INT32_MIN, jnp.int32)
    acc_l, acc_h, _, _ = lax.fori_loop(
        0, 2 * _CHUNK_ITERS, body, (acc0, acc0, inv0, inv0)
    )
    return acc_l, acc_h


def _merge_lanes(acc):
    best = acc[0]
    for j in range(1, _LANES):
        best = jnp.maximum(best, acc[j])
    return best


def _finish_row(rows_v, rr, acc_l, acc_h):
    best_l = _merge_lanes(acc_l)
    best_h = _merge_lanes(acc_h)
    ok = jnp.logical_and(best_l > np.int32(65535), best_h > np.int32(65535))

    def exact(_):
        a_l, a_h = _scan_row_exact(rows_v, rr)
        return _merge_lanes(a_l), _merge_lanes(a_h)

    best_l, best_h = lax.cond(ok, lambda _: (best_l, best_h), exact, None)
    idx_l = np.int32(32767) - (best_l & np.int32(0xFFFF))
    idx_h = np.int32(32767) - (best_h & np.int32(0xFFFF))
    return idx_l, idx_h


@functools.partial(
    pl.kernel,
    mesh=plsc.VectorSubcoreMesh(core_axis_name="c", subcore_axis_name="s"),
    out_type=jax.ShapeDtypeStruct((_NUM_WORKERS, 16), jnp.int32),
    scratch_types=[
        pltpu.VMEM((_PACKED_PER_WORKER, _COLS), jnp.int32),
        pltpu.VMEM((16,), jnp.int32),
        pltpu.SemaphoreType.DMA,
        pltpu.SemaphoreType.DMA,
        pltpu.SemaphoreType.DMA,
        pltpu.SemaphoreType.DMA,
    ],
)
def _sc_argmax(x_hbm, out_hbm, rows_v, res_v, sem0, sem1, sem2, sem3):
    wid = lax.axis_index("s") * _NUM_CORES + lax.axis_index("c")
    x32 = x_hbm.bitcast(jnp.int32)
    base = wid * _PACKED_PER_WORKER
    sems = (sem0, sem1, sem2, sem3)
    copies = []
    for c in range(4):
        rr, hh = divmod(c, 2)
        copies.append(
            pltpu.async_copy(
                x32.at[base + rr, pl.ds(hh * _HALF, _HALF)],
                rows_v.at[rr, pl.ds(hh * _HALF, _HALF)],
                sems[c],
            )
        )
    lane16 = lax.iota(jnp.int32, 16)
    res = jnp.zeros((16,), jnp.int32)
    acc0 = jnp.full((_LANES,), _INT32_MIN, jnp.int32)
    for rr in range(_PACKED_PER_WORKER):
        copies[2 * rr].wait()
        acc = _scan_chunk(rows_v.at[rr, pl.ds(0, _HALF)], 0, (acc0, acc0))
        copies[2 * rr + 1].wait()
        acc = _scan_chunk(rows_v.at[rr, pl.ds(_HALF, _HALF)], _HALF, acc)
        idx_l, idx_h = _finish_row(rows_v, rr, *acc)
        res = jnp.where(lane16 == 2 * rr, idx_l, res)
        res = jnp.where(lane16 == 2 * rr + 1, idx_h, res)
    res_v[...] = res
    pltpu.sync_copy(res_v, out_hbm.at[wid])


def kernel(input_tensor, dim):
    del dim
    out = _sc_argmax(input_tensor)
    return out[:, : 2 * _PACKED_PER_WORKER].reshape(_ROWS).astype(jnp.int64)


# speedup vs baseline: 5.1430x; 1.0258x over previous
import functools

import jax
import jax.numpy as jnp
import numpy as np
from jax import lax
from jax.experimental import pallas as pl
from jax.experimental.pallas import tpu as pltpu
from jax.experimental.pallas import tpu_sc as plsc

_ROWS = 128
_COLS = 32768
_NUM_CORES = 2
_NUM_SUBCORES = 16
_NUM_WORKERS = _NUM_CORES * _NUM_SUBCORES
_PACKED_PER_WORKER = 2
_LANES = 16
_HALF = _COLS // 2
_CHUNK_ITERS = _HALF // _LANES
_ROW_ITERS = 2 * _CHUNK_ITERS

_SIGN2 = np.int32(-2147450880)
_HI16 = np.int32(-65536)
_ONE2 = np.int32(0x00010001)
_INT32_MIN = np.int32(-(2**31))


def _scan_chunk(buf, iter0, carry):
    def body(i, carry):
        acc_l, acc_h, inv = carry
        v = buf[pl.ds(i * _LANES, _LANES)]
        acc_l = jnp.maximum(acc_l, (v << 16) | inv)
        acc_h = jnp.maximum(acc_h, (v & _HI16) | inv)
        return acc_l, acc_h, inv - 1

    inv0 = jnp.full((_LANES,), np.int32(_ROW_ITERS - 1 - iter0), jnp.int32)
    acc_l, acc_h, _ = lax.fori_loop(
        0, _CHUNK_ITERS, body, (*carry, inv0), unroll=8
    )
    return acc_l, acc_h


def _scan_row_exact(rows_v, rr):
    def body(i, carry):
        acc_l, acc_h, inv = carry
        v = rows_v[rr, pl.ds(i * _LANES, _LANES)]
        m = (v & _SIGN2) - ((v >> 15) & _ONE2)
        kk = v ^ m
        acc_l = jnp.maximum(acc_l, (kk << 16) | inv)
        acc_h = jnp.maximum(acc_h, (kk & _HI16) | inv)
        return acc_l, acc_h, inv - 1

    inv0 = jnp.full((_LANES,), np.int32(_ROW_ITERS - 1), jnp.int32)
    acc0 = jnp.full((_LANES,), _INT32_MIN, jnp.int32)
    acc_l, acc_h, _ = lax.fori_loop(
        0, _ROW_ITERS, body, (acc0, acc0, inv0)
    )
    return acc_l, acc_h


def _merge_lanes(acc):
    best = acc[0]
    lane = np.int32(0)
    for j in range(1, _LANES):
        a = acc[j]
        upd = a > best
        best = jnp.where(upd, a, best)
        lane = jnp.where(upd, np.int32(j), lane)
    return best, lane


def _to_col(best, lane):
    it = np.int32(_ROW_ITERS - 1) - (best & np.int32(0xFFFF))
    return it * _LANES + lane


def _finish_row(rows_v, rr, acc_l, acc_h):
    best_l, lane_l = _merge_lanes(acc_l)
    best_h, lane_h = _merge_lanes(acc_h)
    ok = jnp.logical_and(best_l > np.int32(65535), best_h > np.int32(65535))

    def fast(_):
        return _to_col(best_l, lane_l), _to_col(best_h, lane_h)

    def exact(_):
        a_l, a_h = _scan_row_exact(rows_v, rr)
        b_l, ln_l = _merge_lanes(a_l)
        b_h, ln_h = _merge_lanes(a_h)
        return _to_col(b_l, ln_l), _to_col(b_h, ln_h)

    return lax.cond(ok, fast, exact, None)


@functools.partial(
    pl.kernel,
    mesh=plsc.VectorSubcoreMesh(core_axis_name="c", subcore_axis_name="s"),
    out_type=jax.ShapeDtypeStruct((_NUM_WORKERS, 16), jnp.int32),
    scratch_types=[
        pltpu.VMEM((_PACKED_PER_WORKER, _COLS), jnp.int32),
        pltpu.VMEM((16,), jnp.int32),
        pltpu.SemaphoreType.DMA,
        pltpu.SemaphoreType.DMA,
        pltpu.SemaphoreType.DMA,
        pltpu.SemaphoreType.DMA,
    ],
)
def _sc_argmax(x_hbm, out_hbm, rows_v, res_v, sem0, sem1, sem2, sem3):
    wid = lax.axis_index("s") * _NUM_CORES + lax.axis_index("c")
    x32 = x_hbm.bitcast(jnp.int32)
    base = wid * _PACKED_PER_WORKER
    sems = (sem0, sem1, sem2, sem3)
    copies = []
    for c in range(4):
        rr, hh = divmod(c, 2)
        copies.append(
            pltpu.async_copy(
                x32.at[base + rr, pl.ds(hh * _HALF, _HALF)],
                rows_v.at[rr, pl.ds(hh * _HALF, _HALF)],
                sems[c],
            )
        )
    lane16 = lax.iota(jnp.int32, 16)
    res = jnp.zeros((16,), jnp.int32)
    acc0 = jnp.full((_LANES,), _INT32_MIN, jnp.int32)
    for rr in range(_PACKED_PER_WORKER):
        copies[2 * rr].wait()
        acc = _scan_chunk(rows_v.at[rr, pl.ds(0, _HALF)], 0, (acc0, acc0))
        copies[2 * rr + 1].wait()
        acc = _scan_chunk(
            rows_v.at[rr, pl.ds(_HALF, _HALF)], _CHUNK_ITERS, acc
        )
        idx_l, idx_h = _finish_row(rows_v, rr, *acc)
        res = jnp.where(lane16 == 2 * rr, idx_l, res)
        res = jnp.where(lane16 == 2 * rr + 1, idx_h, res)
    res_v[...] = res
    pltpu.sync_copy(res_v, out_hbm.at[wid])


def kernel(input_tensor, dim):
    del dim
    out = _sc_argmax(input_tensor)
    return out[:, : 2 * _PACKED_PER_WORKER].reshape(_ROWS).astype(jnp.int64)
